# baseline (speedup 1.0000x reference)
import functools

import jax
import jax.numpy as jnp
from jax import lax
from jax.experimental import pallas as pl
from jax.experimental.pallas import tpu as pltpu
from jax.experimental.pallas import tpu_sc as plsc

VOCAB = 100000
D = 128
B = 16384
NC = 2
NS = 16
NW = NC * NS
PER_W = B // NW
CHUNK = 128
NCHUNK = PER_W // CHUNK
L = 16
GROUPS = CHUNK // L


def _sc_body(ti_hbm, cj_hbm, table_hbm, out_hbm,
             idx_t, idx_c, trows, crows, outv, sem_t, sem_c):
    wid = lax.axis_index("s") * NC + lax.axis_index("c")
    base = wid * PER_W

    for j in range(NCHUNK):
        pltpu.sync_copy(ti_hbm.at[pl.ds(base + j * CHUNK, CHUNK)], idx_t.at[j])
        pltpu.sync_copy(cj_hbm.at[pl.ds(base + j * CHUNK, CHUNK)], idx_c.at[j])

    for j in range(NCHUNK):
        cp_t = pltpu.async_copy(table_hbm.at[idx_t.at[j]], trows, sem_t)
        cp_c = pltpu.async_copy(table_hbm.at[idx_c.at[j]], crows, sem_c)
        cp_t.wait()
        cp_c.wait()

        lane = lax.iota(jnp.int32, L)

        def group_body(g, carry, j=j, lane=lane):
            tot = jnp.zeros((L,), jnp.float32)
            for p in range(L):
                r = g * L + p
                acc = trows[r, pl.ds(0, L)] * crows[r, pl.ds(0, L)]
                for k in range(1, D // L):
                    acc = acc + trows[r, pl.ds(k * L, L)] * crows[r, pl.ds(k * L, L)]
                tot = jnp.where(lane == p, jnp.sum(acc), tot)
            outv[pl.ds(j * CHUNK + g * L, L)] = 1.0 / (1.0 + jnp.exp(-tot))
            return carry

        lax.fori_loop(0, GROUPS, group_body, 0)

    pltpu.sync_copy(outv, out_hbm.at[pl.ds(base, PER_W)])


_sc_kernel = functools.partial(
    pl.kernel,
    out_type=jax.ShapeDtypeStruct((B,), jnp.float32),
    mesh=plsc.VectorSubcoreMesh(core_axis_name="c", subcore_axis_name="s",
                                num_cores=NC, num_subcores=NS),
    compiler_params=pltpu.CompilerParams(needs_layout_passes=False),
    scratch_types=[
        pltpu.VMEM((NCHUNK, CHUNK), jnp.int32),
        pltpu.VMEM((NCHUNK, CHUNK), jnp.int32),
        pltpu.VMEM((CHUNK, D), jnp.float32),
        pltpu.VMEM((CHUNK, D), jnp.float32),
        pltpu.VMEM((PER_W,), jnp.float32),
        pltpu.SemaphoreType.DMA,
        pltpu.SemaphoreType.DMA,
    ],
)(_sc_body)


def kernel(target_i, context_j, shared_embedding):
    ti = target_i.astype(jnp.int32)
    cj = context_j.astype(jnp.int32)
    return _sc_kernel(ti, cj, shared_embedding)
